# baseline (speedup 1.0000x reference)
import functools

import jax
import jax.numpy as jnp
from jax import lax
from jax.experimental import pallas as pl
from jax.experimental.pallas import tpu as pltpu
from jax.experimental.pallas import tpu_sc as plsc

D_MODEL = 2048
BATCH = 4
SEQ = 4096

_info = plsc.get_sparse_core_info()
NC, NS = _info.num_cores, _info.num_subcores
NW = NC * NS

POS_PER_W = SEQ // NW
CHUNK = 8
STEPS = POS_PER_W // CHUNK
NBUF = 6
LANES = 16

_mesh = plsc.VectorSubcoreMesh(core_axis_name="c", subcore_axis_name="s")


@functools.partial(
    pl.kernel,
    out_type=jax.ShapeDtypeStruct((BATCH * SEQ, D_MODEL), jnp.float32),
    mesh=_mesh,
    scratch_types=[
        pltpu.VMEM((BATCH, POS_PER_W), jnp.int32),
        pltpu.VMEM((NBUF, CHUNK, D_MODEL), jnp.float32),
        pltpu.VMEM((CHUNK, D_MODEL), jnp.float32),
        pltpu.SemaphoreType.DMA((NBUF,)),
        pltpu.SemaphoreType.DMA,
        pltpu.SemaphoreType.DMA((NBUF,)),
    ],
)
def _emb_kernel(x_hbm, table_hbm, pe_hbm, out_hbm, idx_v, rows, pebuf,
                gsem, psem, wsem):
    wid = lax.axis_index("s") * NC + lax.axis_index("c")
    pos0 = wid * POS_PER_W

    for b in range(BATCH):
        pltpu.sync_copy(x_hbm.at[pl.ds(b * SEQ + pos0, POS_PER_W)],
                        idx_v.at[b])

    def buf(g, b):
        return lax.rem(4 * g + b, NBUF)

    def pe_copy(g):
        return pltpu.make_async_copy(
            pe_hbm.at[pl.ds(pos0 + g * CHUNK, CHUNK)], pebuf, psem)

    def gather_copy(g, b):
        m = buf(g, b)
        return pltpu.make_async_copy(
            table_hbm.at[idx_v.at[b, pl.ds(g * CHUNK, CHUNK)]],
            rows.at[m], gsem.at[m])

    def out_copy(g, b):
        m = buf(g, b)
        return pltpu.make_async_copy(
            rows.at[m],
            out_hbm.at[pl.ds(b * SEQ + pos0 + g * CHUNK, CHUNK)],
            wsem.at[m])

    pe_copy(0).start()
    for b in range(BATCH):
        gather_copy(0, b).start()

    def step(g, first, last):
        for b in range(BATCH):
            if b == 0:
                pe_copy(g).wait()
            m = buf(g, b)
            gather_copy(g, b).wait()
            for row in range(CHUNK):
                @plsc.parallel_loop(0, D_MODEL, LANES, unroll=16)
                def _add(l, row=row, m=m):
                    plsc.addupdate(
                        rows.at[m, row, pl.ds(l, LANES)],
                        pebuf[row, pl.ds(l, LANES)])
            if b == BATCH - 1 and not last:
                pe_copy(g + 1).start()
            out_copy(g, b).start()
            prv = (g - 1, b + 2) if b < 2 else (g, b - 2)
            if not last:
                if not (first and b < 2):
                    out_copy(*prv).wait()
                gather_copy(g + 1, b).start()

    step(0, True, False)
    pl.loop(1, STEPS - 1)(lambda g: step(g, False, False))
    step(STEPS - 1, False, True)

    for b in (2, 3):
        out_copy(STEPS - 2, b).wait()
    for b in range(BATCH):
        out_copy(STEPS - 1, b).wait()


def kernel(x, table, pe):
    flat = _emb_kernel(x.reshape(-1), table, pe)
    return flat.reshape(BATCH, SEQ, D_MODEL)


# speedup vs baseline: 1.1258x; 1.1258x over previous
import functools

import jax
import jax.numpy as jnp
from jax import lax
from jax.experimental import pallas as pl
from jax.experimental.pallas import tpu as pltpu
from jax.experimental.pallas import tpu_sc as plsc

D_MODEL = 2048
BATCH = 4
SEQ = 4096

_info = plsc.get_sparse_core_info()
NC, NS = _info.num_cores, _info.num_subcores
NW = NC * NS

POS_PER_W = SEQ // NW
CHUNK = 8
STEPS = POS_PER_W // CHUNK
NBUF = 5
LANES = 16

_mesh = plsc.VectorSubcoreMesh(core_axis_name="c", subcore_axis_name="s")


@functools.partial(
    pl.kernel,
    out_type=jax.ShapeDtypeStruct((BATCH * SEQ, D_MODEL), jnp.float32),
    mesh=_mesh,
    scratch_types=[
        pltpu.VMEM((BATCH, POS_PER_W), jnp.int32),
        pltpu.VMEM((NBUF, CHUNK, D_MODEL), jnp.float32),
        pltpu.VMEM((2, CHUNK, D_MODEL), jnp.float32),
        pltpu.SemaphoreType.DMA((NBUF,)),
        pltpu.SemaphoreType.DMA((2,)),
        pltpu.SemaphoreType.DMA((NBUF,)),
    ],
)
def _emb_kernel(x_hbm, table_hbm, pe_hbm, out_hbm, idx_v, rows, pebuf,
                gsem, psem, wsem):
    wid = lax.axis_index("s") * NC + lax.axis_index("c")
    pos0 = wid * POS_PER_W

    for b in range(BATCH):
        pltpu.sync_copy(x_hbm.at[pl.ds(b * SEQ + pos0, POS_PER_W)],
                        idx_v.at[b])

    def buf(g, b):
        return lax.rem(4 * g + b, NBUF)

    def pe_copy(g, slot):
        return pltpu.make_async_copy(
            pe_hbm.at[pl.ds(pos0 + g * CHUNK, CHUNK)],
            pebuf.at[slot], psem.at[slot])

    def gather_copy(g, b):
        m = buf(g, b)
        return pltpu.make_async_copy(
            table_hbm.at[idx_v.at[b, pl.ds(g * CHUNK, CHUNK)]],
            rows.at[m], gsem.at[m])

    def out_copy(g, b):
        m = buf(g, b)
        return pltpu.make_async_copy(
            rows.at[m],
            out_hbm.at[pl.ds(b * SEQ + pos0 + g * CHUNK, CHUNK)],
            wsem.at[m])

    pe_copy(0, 0).start()
    for b in range(3):
        gather_copy(0, b).start()

    def step(g, first, last):
        slot = lax.rem(g, 2)
        for b in range(BATCH):
            if b == 0:
                pe_copy(g, slot).wait()
                if not last:
                    pe_copy(g + 1, 1 - slot).start()
            m = buf(g, b)
            gather_copy(g, b).wait()
            nxt = (g, 3) if b == 0 else (g + 1, b - 1)
            prv = (g - 1, b + 2) if b < 2 else (g, b - 2)
            if not (last and b > 0):
                if not (first and b < 2):
                    out_copy(*prv).wait()
                gather_copy(*nxt).start()
            for row in range(CHUNK):
                @plsc.parallel_loop(0, D_MODEL, LANES, unroll=16)
                def _add(l, row=row, m=m, slot=slot):
                    plsc.addupdate(
                        rows.at[m, row, pl.ds(l, LANES)],
                        pebuf[slot, row, pl.ds(l, LANES)])
            out_copy(g, b).start()

    step(0, True, False)
    pl.loop(1, STEPS - 1)(lambda g: step(g, False, False))
    step(STEPS - 1, False, True)

    out_copy(STEPS - 2, 3).wait()
    for b in range(BATCH):
        out_copy(STEPS - 1, b).wait()


def kernel(x, table, pe):
    flat = _emb_kernel(x.reshape(-1), table, pe)
    return flat.reshape(BATCH, SEQ, D_MODEL)
